# Initial kernel scaffold; baseline (speedup 1.0000x reference)
import functools

import jax
import jax.numpy as jnp
from jax import lax
from jax.experimental import pallas as pl
from jax.experimental.pallas import tpu as pltpu
from jax.experimental.pallas import tpu_sc as plsc

NC, NS = 2, 16
NW = NC * NS
CH = 128


def _xw_body(wc_ref, w_ref, x_ref, o_ref):
    r = pl.program_id(1)
    w = w_ref[...]
    wrel = wc_ref[r, 0] * w[0]
    for b in range(1, w.shape[0]):
        wrel = wrel + wc_ref[r, b] * w[b]
    o_ref[...] = jnp.dot(x_ref[...], wrel, preferred_element_type=jnp.float32)


def _add_body(p_ref, o_ref):
    o_ref[...] = p_ref[0] + p_ref[1]


def kernel(x, edge_index, edge_type, weight, w_comp):
    N, F = x.shape
    B, _, O = weight.shape
    R = w_comp.shape[0]
    E = edge_index.shape[1]

    BN = 2000
    NB = N // BN
    xw = pl.pallas_call(
        _xw_body,
        grid=(NB, R),
        in_specs=[
            pl.BlockSpec(memory_space=pltpu.SMEM),
            pl.BlockSpec((B, F, O), lambda n, r: (0, 0, 0)),
            pl.BlockSpec((BN, F), lambda n, r: (n, 0)),
        ],
        out_specs=pl.BlockSpec((BN, O), lambda n, r: (r * NB + n, 0)),
        out_shape=jax.ShapeDtypeStruct((R * N, O), jnp.float32),
    )(w_comp, weight, x)

    src = edge_index[0]
    dst = edge_index[1]
    gidx = edge_type * N + src
    NCHUNK = -(-E // (NW * CH))
    e_pad = NW * NCHUNK * CH
    pad = e_pad - E
    gidx = jnp.concatenate([gidx, jnp.zeros((pad,), jnp.int32)])
    dstp = jnp.concatenate([dst, jnp.full((pad,), N, jnp.int32)])
    gidx = gidx.reshape(NW * NCHUNK, CH)
    dstp = dstp.reshape(NW * NCHUNK, CH)

    ZR = -(-(N + 1) // NS)
    N_pad = ZR * NS
    zrows = jnp.zeros((ZR, O), jnp.float32)

    mesh = plsc.VectorSubcoreMesh(
        core_axis_name="c", subcore_axis_name="s", num_cores=NC, num_subcores=NS
    )

    @functools.partial(
        pl.kernel,
        out_type=jax.ShapeDtypeStruct((NC, N_pad, O), jnp.float32),
        mesh=mesh,
        scratch_types=[
            pltpu.VMEM((CH,), jnp.int32),
            pltpu.VMEM((CH,), jnp.int32),
            pltpu.VMEM((CH, O), jnp.float32),
            pltpu.VMEM_SHARED((N_pad, O), jnp.float32),
            pltpu.SemaphoreType.DMA,
        ],
    )
    def sc_scatter(xw_hbm, gidx_hbm, dst_hbm, z_hbm, out_hbm,
                   idx_v, dst_v, rows_v, acc, gsem):
        cid = lax.axis_index("c")
        sid = lax.axis_index("s")
        wid = cid * NS + sid
        pltpu.sync_copy(z_hbm, acc.at[pl.ds(sid * ZR, ZR)])
        plsc.subcore_barrier()
        base = wid * NCHUNK

        def body(i, carry):
            row = base + i
            pltpu.sync_copy(gidx_hbm.at[row], idx_v)
            pltpu.sync_copy(dst_hbm.at[row], dst_v)
            pltpu.async_copy(xw_hbm.at[idx_v], rows_v, gsem).wait()
            pltpu.sync_copy(rows_v, acc.at[dst_v], add=True)
            return carry

        lax.fori_loop(0, NCHUNK, body, 0)
        plsc.subcore_barrier()
        pltpu.sync_copy(acc.at[pl.ds(sid * ZR, ZR)],
                        out_hbm.at[cid, pl.ds(sid * ZR, ZR)])

    partial = sc_scatter(xw, gidx, dstp, zrows)

    BN2 = 1000
    out = pl.pallas_call(
        _add_body,
        grid=(N // BN2,),
        in_specs=[pl.BlockSpec((NC, BN2, O), lambda n: (0, n, 0))],
        out_specs=pl.BlockSpec((BN2, O), lambda n: (n, 0)),
        out_shape=jax.ShapeDtypeStruct((N, O), jnp.float32),
    )(partial)
    return out


# speedup vs baseline: 12.6053x; 12.6053x over previous
import functools

import jax
import jax.numpy as jnp
from jax import lax
from jax.experimental import pallas as pl
from jax.experimental.pallas import tpu as pltpu
from jax.experimental.pallas import tpu_sc as plsc

NC, NS = 2, 16
NW = NC * NS
CH = 128


def _xw_body(wc_ref, w_ref, x_ref, o_ref):
    r = pl.program_id(1)
    w = w_ref[...]
    wrel = wc_ref[r, 0] * w[0]
    for b in range(1, w.shape[0]):
        wrel = wrel + wc_ref[r, b] * w[b]
    o_ref[...] = jnp.dot(x_ref[...], wrel, preferred_element_type=jnp.float32)


def _add_body(p_ref, o_ref):
    o_ref[...] = p_ref[0] + p_ref[1]


def kernel(x, edge_index, edge_type, weight, w_comp):
    N, F = x.shape
    B, _, O = weight.shape
    R = w_comp.shape[0]
    E = edge_index.shape[1]

    BN = 2000
    NB = N // BN
    xw = pl.pallas_call(
        _xw_body,
        grid=(NB, R),
        in_specs=[
            pl.BlockSpec(memory_space=pltpu.SMEM),
            pl.BlockSpec((B, F, O), lambda n, r: (0, 0, 0)),
            pl.BlockSpec((BN, F), lambda n, r: (n, 0)),
        ],
        out_specs=pl.BlockSpec((BN, O), lambda n, r: (r * NB + n, 0)),
        out_shape=jax.ShapeDtypeStruct((R * N, O), jnp.float32),
    )(w_comp, weight, x)

    src = edge_index[0]
    dst = edge_index[1]
    gidx = edge_type * N + src
    NCHUNK = -(-E // (NW * CH))
    e_pad = NW * NCHUNK * CH
    pad = e_pad - E
    gidx = jnp.concatenate([gidx, jnp.zeros((pad,), jnp.int32)])
    dstp = jnp.concatenate([dst, jnp.full((pad,), N, jnp.int32)])
    gidx = gidx.reshape(NW * NCHUNK, CH)
    dstp = dstp.reshape(NW * NCHUNK, CH)

    ZR = (-(-(N + 1) // NS) + 7) // 8 * 8
    N_pad = ZR * NS
    zrows = jnp.zeros((ZR, O), jnp.float32)

    mesh = plsc.VectorSubcoreMesh(
        core_axis_name="c", subcore_axis_name="s", num_cores=NC, num_subcores=NS
    )

    @functools.partial(
        pl.kernel,
        out_type=jax.ShapeDtypeStruct((NC, N_pad, O), jnp.float32),
        mesh=mesh,
        scratch_types=[
            pltpu.VMEM((CH,), jnp.int32),
            pltpu.VMEM((CH,), jnp.int32),
            pltpu.VMEM((CH, O), jnp.float32),
            pltpu.VMEM_SHARED((N_pad, O), jnp.float32),
            pltpu.SemaphoreType.DMA,
        ],
    )
    def sc_scatter(xw_hbm, gidx_hbm, dst_hbm, z_hbm, out_hbm,
                   idx_v, dst_v, rows_v, acc, gsem):
        cid = lax.axis_index("c")
        sid = lax.axis_index("s")
        wid = cid * NS + sid
        pltpu.sync_copy(z_hbm, acc.at[pl.ds(sid * ZR, ZR)])
        plsc.subcore_barrier()
        base = wid * NCHUNK

        def body(i, carry):
            row = base + i
            pltpu.sync_copy(gidx_hbm.at[row], idx_v)
            pltpu.sync_copy(dst_hbm.at[row], dst_v)
            pltpu.async_copy(xw_hbm.at[idx_v], rows_v, gsem).wait()
            pltpu.sync_copy(rows_v, acc.at[dst_v], add=True)
            return carry

        lax.fori_loop(0, NCHUNK, body, 0)
        plsc.subcore_barrier()
        pltpu.sync_copy(acc.at[pl.ds(sid * ZR, ZR)],
                        out_hbm.at[cid, pl.ds(sid * ZR, ZR)])

    partial = sc_scatter(xw, gidx, dstp, zrows)

    BN2 = 1000
    out = pl.pallas_call(
        _add_body,
        grid=(N // BN2,),
        in_specs=[pl.BlockSpec((NC, BN2, O), lambda n: (0, n, 0))],
        out_specs=pl.BlockSpec((BN2, O), lambda n: (n, 0)),
        out_shape=jax.ShapeDtypeStruct((N, O), jnp.float32),
    )(partial)
    return out
